# Initial kernel scaffold; baseline (speedup 1.0000x reference)
import jax
import jax.numpy as jnp
from jax.experimental import pallas as pl


def kernel(item_name, text_description, brand_name, category, item_condition_id, shipping_flag, cat1, cat2, cat3, name_table, text_table, brand_table, category_table, condition_table, cat1_table, cat2_table, cat3_table, bn1_gamma, bn1_beta, bn1_mean, bn1_var, fc1_w, fc1_b, bn2_gamma, bn2_beta, bn2_mean, bn2_var, fc2_w, fc2_b):
    raise NotImplementedError("write your pallas kernel here")


# speedup vs baseline: 5.3271x; 5.3271x over previous
import jax
import jax.numpy as jnp
from jax import lax
from jax.experimental import pallas as pl
from jax.experimental.pallas import tpu as pltpu
from jax.experimental.pallas import tpu_sc as plsc

B = 16384
EPS = 1e-5
NC, NS = 2, 16
NW = NC * NS
IPT = B // NW

N_CHUNK_ITEMS, N_IDX = 4, 80
T_CHUNK_ITEMS, T_IDX = 2, 100


def _sc_embed(nidx, tidx, bidx, cidx, coidx, c1i, c2i, c3i,
              ntab, ttab, btab, ctab, cotab, t1, t2, t3):
    mesh = plsc.VectorSubcoreMesh(core_axis_name="c", subcore_axis_name="s")
    f32 = jnp.float32

    out_type = [
        jax.ShapeDtypeStruct((B, 128), f32),
        jax.ShapeDtypeStruct((B, 128), f32),
        jax.ShapeDtypeStruct((B, 64), f32),
        jax.ShapeDtypeStruct((B, 32), f32),
        jax.ShapeDtypeStruct((B, 16), f32),
        jax.ShapeDtypeStruct((B, 16), f32),
        jax.ShapeDtypeStruct((B, 16), f32),
        jax.ShapeDtypeStruct((B, 16), f32),
    ]
    scratch_types = [
        pltpu.VMEM((128, N_IDX), jnp.int32),
        pltpu.VMEM((256, T_IDX), jnp.int32),
        pltpu.VMEM((4, 128), jnp.int32),
        pltpu.VMEM((N_CHUNK_ITEMS * 20, 128), f32),
        pltpu.VMEM((T_CHUNK_ITEMS * 50, 128), f32),
        pltpu.VMEM((128, 64), f32),
        pltpu.VMEM((128, 32), f32),
        pltpu.VMEM((128, 16), f32),
        pltpu.VMEM((32, 128), f32),
        pltpu.VMEM((32, 128), f32),
    ]

    @jax.named_scope("sc_embed")
    def body(nidx_r, tidx_r, bidx_r, cidx_r, coidx_r, c1i_r, c2i_r, c3i_r,
             ntab_r, ttab_r, btab_r, ctab_r, cotab_r, t1_r, t2_r, t3_r,
             name_o, text_o, brand_o, cat_o, cond_o, c1o, c2o, c3o,
             nidx_v, tidx_v, sidx_v, gn, gt, gs64, gs32, gs16, st_n, st_t):
        wid = lax.axis_index("s") * NC + lax.axis_index("c")
        base = wid * IPT

        pltpu.sync_copy(nidx_r.at[pl.ds(wid * 128, 128)], nidx_v)

        @pl.loop(0, 16)
        def _(jo):
            for ji in range(8):
                j = jo * 8 + ji
                pltpu.sync_copy(ntab_r.at[nidx_v.at[j]], gn)

                @pl.loop(0, N_CHUNK_ITEMS)
                def _(it):
                    @pl.loop(0, 8)
                    def _(c):
                        cs = pl.ds(c * 16, 16)
                        acc = gn[it * 20, cs]
                        for r in range(1, 20):
                            acc = acc + gn[it * 20 + r, cs]
                        st_n[ji * 4 + it, cs] = acc * (1.0 / 20.0)

            pltpu.sync_copy(st_n, name_o.at[pl.ds(base + jo * 32, 32)])

        pltpu.sync_copy(tidx_r.at[pl.ds(wid * 256, 256)], tidx_v)

        @pl.loop(0, 16)
        def _(jo):
            for ji in range(16):
                j = jo * 16 + ji
                pltpu.sync_copy(ttab_r.at[tidx_v.at[j]], gt)

                @pl.loop(0, T_CHUNK_ITEMS)
                def _(it):
                    @pl.loop(0, 8)
                    def _(c):
                        cs = pl.ds(c * 16, 16)
                        acc = gt[it * 50, cs]
                        for r in range(1, 50):
                            acc = acc + gt[it * 50 + r, cs]
                        st_t[ji * 2 + it, cs] = acc * (1.0 / 50.0)

            pltpu.sync_copy(st_t, text_o.at[pl.ds(base + jo * 32, 32)])

        def plain(idx2d, tab, out, gdst):
            pltpu.sync_copy(idx2d.at[pl.ds(wid * 4, 4)], sidx_v)

            @pl.loop(0, 4)
            def _(k):
                pltpu.sync_copy(tab.at[sidx_v.at[k]], gdst)
                pltpu.sync_copy(gdst, out.at[pl.ds(base + k * 128, 128)])

        plain(bidx_r, btab_r, brand_o, gs64)
        plain(cidx_r, ctab_r, cat_o, gs32)
        plain(coidx_r, cotab_r, cond_o, gs16)
        plain(c1i_r, t1_r, c1o, gs16)
        plain(c2i_r, t2_r, c2o, gs16)
        plain(c3i_r, t3_r, c3o, gs16)

    run = pl.kernel(body, out_type=out_type, mesh=mesh,
                    scratch_types=scratch_types,
                    compiler_params=pltpu.CompilerParams(
                        use_tc_tiling_on_sc=False))
    return run(nidx, tidx, bidx, cidx, coidx, c1i, c2i, c3i,
               ntab, ttab, btab, ctab, cotab, t1, t2, t3)


def _tc_mlp(name_e, text_e, brand_e, cat_e, cond_e, ship, c1_e, c2_e, c3_e,
            w1, b1, g1, be1, m1, v1, g2, be2, m2, v2, w2, b2):
    BLK = 2048
    grid = (B // BLK,)

    def body(n_r, t_r, br_r, ca_r, co_r, sh_r, x1_r, x2_r, x3_r,
             w1_r, b1_r, g1_r, be1_r, m1_r, v1_r,
             g2_r, be2_r, m2_r, v2_r, w2_r, b2_r, out_r):
        w1f = w1_r[...]
        s1 = g1_r[...] * lax.rsqrt(v1_r[...] + EPS)
        w1s = w1f * s1.reshape(417, 1)
        b1f = b1_r[...] + (be1_r[...] - m1_r[...] * s1) @ w1f

        f32 = jnp.float32
        z = jnp.dot(n_r[...], w1s[0:128], preferred_element_type=f32)
        z += jnp.dot(t_r[...], w1s[128:256], preferred_element_type=f32)
        z += jnp.dot(br_r[...], w1s[256:320], preferred_element_type=f32)
        z += jnp.dot(ca_r[...], w1s[320:352], preferred_element_type=f32)
        z += jnp.dot(co_r[...], w1s[352:368], preferred_element_type=f32)
        z += sh_r[...] * w1s[368:369]
        z += jnp.dot(x1_r[...], w1s[369:385], preferred_element_type=f32)
        z += jnp.dot(x2_r[...], w1s[385:401], preferred_element_type=f32)
        z += jnp.dot(x3_r[...], w1s[401:417], preferred_element_type=f32)
        z += b1f
        h = jnp.where(z > 0, z, 0.01 * z)

        w2f = w2_r[...]
        s2 = g2_r[...] * lax.rsqrt(v2_r[...] + EPS)
        w2s = w2f * s2.reshape(150, 1)
        b2f = b2_r[...] + (be2_r[...] - m2_r[...] * s2) @ w2f
        out_r[...] = jnp.dot(h, w2s, preferred_element_type=f32) + b2f

    row_spec = lambda w: pl.BlockSpec((BLK, w), lambda i: (i, 0))
    full = lambda a: pl.BlockSpec(a.shape, lambda i: (0,) * a.ndim)

    return pl.pallas_call(
        body,
        grid=grid,
        in_specs=[
            row_spec(128), row_spec(128), row_spec(64), row_spec(32),
            row_spec(16), row_spec(1), row_spec(16), row_spec(16),
            row_spec(16),
            full(w1), full(b1), full(g1), full(be1), full(m1), full(v1),
            full(g2), full(be2), full(m2), full(v2), full(w2), full(b2),
        ],
        out_specs=pl.BlockSpec((BLK, 1), lambda i: (i, 0)),
        out_shape=jax.ShapeDtypeStruct((B, 1), jnp.float32),
    )(name_e, text_e, brand_e, cat_e, cond_e, ship, c1_e, c2_e, c3_e,
      w1, b1, g1, be1, m1, v1, g2, be2, m2, v2, w2, b2)


def kernel(item_name, text_description, brand_name, category,
           item_condition_id, shipping_flag, cat1, cat2, cat3,
           name_table, text_table, brand_table, category_table,
           condition_table, cat1_table, cat2_table, cat3_table,
           bn1_gamma, bn1_beta, bn1_mean, bn1_var, fc1_w, fc1_b,
           bn2_gamma, bn2_beta, bn2_mean, bn2_var, fc2_w, fc2_b):
    i32 = jnp.int32
    nidx = item_name.astype(i32).reshape(B * 20 // N_IDX, N_IDX)
    tidx = text_description.astype(i32).reshape(B * 50 // T_IDX, T_IDX)
    bidx = brand_name.astype(i32).reshape(128, 128)
    cidx = category.astype(i32).reshape(128, 128)
    coidx = item_condition_id.astype(i32).reshape(128, 128)
    c1i = cat1.astype(i32).reshape(128, 128)
    c2i = cat2.astype(i32).reshape(128, 128)
    c3i = cat3.astype(i32).reshape(128, 128)

    name_e, text_e, brand_e, cat_e, cond_e, c1_e, c2_e, c3_e = _sc_embed(
        nidx, tidx, bidx, cidx, coidx, c1i, c2i, c3i,
        name_table, text_table, brand_table, category_table,
        condition_table, cat1_table, cat2_table, cat3_table)

    r1 = lambda a: a.reshape(1, -1)
    return _tc_mlp(
        name_e, text_e, brand_e, cat_e, cond_e, shipping_flag,
        c1_e, c2_e, c3_e,
        fc1_w, r1(fc1_b), r1(bn1_gamma), r1(bn1_beta), r1(bn1_mean),
        r1(bn1_var), r1(bn2_gamma), r1(bn2_beta), r1(bn2_mean),
        r1(bn2_var), fc2_w, r1(fc2_b))
